# initial kernel scaffold (unmeasured)
import jax
import jax.numpy as jnp
from jax import lax
from jax.experimental import pallas as pl
from jax.experimental.pallas import tpu as pltpu


def kernel(
    x,
):
    def body(*refs):
        pass

    out_shape = jax.ShapeDtypeStruct(..., jnp.float32)
    return pl.pallas_call(body, out_shape=out_shape)(...)


# baseline (device time: 81729 ns/iter reference)
import jax
import jax.numpy as jnp
from jax import lax
from jax.experimental import pallas as pl
from jax.experimental.pallas import tpu as pltpu

Z = 4
M = 1024
N = 2048
NP = N // Z


def kernel(x):
    def body(x_ref, out_ref, comm_ref, send_sems, recv_sems):
        mx = lax.axis_index("x")
        my = lax.axis_index("y")
        mz = lax.axis_index("z")
        left = (mz + Z - 1) % Z
        right = (mz + 1) % Z

        barrier_sem = pltpu.get_barrier_semaphore()
        for nbr in (left, right):
            pl.semaphore_signal(
                barrier_sem,
                inc=1,
                device_id=(mx, my, nbr),
                device_id_type=pl.DeviceIdType.MESH,
            )
        pl.semaphore_wait(barrier_sem, 2)

        for s in range(Z - 1):
            send_j = (mz + Z - 1 - s) % Z
            if s == 0:
                src = x_ref.at[0, :, pl.ds(send_j * NP, NP)]
            else:
                src = comm_ref.at[s - 1]
            rdma = pltpu.make_async_remote_copy(
                src_ref=src,
                dst_ref=comm_ref.at[s],
                send_sem=send_sems.at[s],
                recv_sem=recv_sems.at[s],
                device_id=(mx, my, right),
                device_id_type=pl.DeviceIdType.MESH,
            )
            rdma.start()
            rdma.wait()
            recv_j = (mz + Z - 2 - s) % Z
            comm_ref[s] = comm_ref[s] + x_ref[0, :, pl.ds(recv_j * NP, NP)]

        out_ref[:, :] = comm_ref[Z - 2]

    return pl.pallas_call(
        body,
        out_shape=jax.ShapeDtypeStruct((M, NP), jnp.float32),
        in_specs=[pl.BlockSpec(memory_space=pltpu.VMEM)],
        out_specs=pl.BlockSpec(memory_space=pltpu.VMEM),
        scratch_shapes=[
            pltpu.VMEM((Z - 1, M, NP), jnp.float32),
            pltpu.SemaphoreType.DMA((Z - 1,)),
            pltpu.SemaphoreType.DMA((Z - 1,)),
        ],
        compiler_params=pltpu.CompilerParams(collective_id=0),
    )(x)


# device time: 61294 ns/iter; 1.3334x vs baseline; 1.3334x over previous
import jax
import jax.numpy as jnp
from jax import lax
from jax.experimental import pallas as pl
from jax.experimental.pallas import tpu as pltpu

Z = 4
M = 1024
MH = M // 2
N = 2048
NP = N // Z


def kernel(x):
    def body(x_ref, out_ref, comm_ref, send_sems, recv_sems, x_sem_s, x_sem_r):
        mx = lax.axis_index("x")
        my = lax.axis_index("y")
        mz = lax.axis_index("z")
        left = (mz + Z - 1) % Z
        right = (mz + 1) % Z
        ro = mx * MH

        barrier_sem = pltpu.get_barrier_semaphore()
        pl.semaphore_signal(
            barrier_sem, inc=1,
            device_id=(mx, my, left), device_id_type=pl.DeviceIdType.MESH,
        )
        pl.semaphore_signal(
            barrier_sem, inc=1,
            device_id=(mx, my, right), device_id_type=pl.DeviceIdType.MESH,
        )
        pl.semaphore_signal(
            barrier_sem, inc=1,
            device_id=(1 - mx, my, mz), device_id_type=pl.DeviceIdType.MESH,
        )
        pl.semaphore_wait(barrier_sem, 3)

        for s in range(Z - 1):
            send_j = (mz + Z - 1 - s) % Z
            if s == 0:
                src = x_ref.at[0, pl.ds(ro, MH), pl.ds(send_j * NP, NP)]
            else:
                src = comm_ref.at[s - 1]
            rdma = pltpu.make_async_remote_copy(
                src_ref=src,
                dst_ref=comm_ref.at[s],
                send_sem=send_sems.at[s],
                recv_sem=recv_sems.at[s],
                device_id=(mx, my, right),
                device_id_type=pl.DeviceIdType.MESH,
            )
            rdma.start()
            rdma.wait()
            recv_j = (mz + Z - 2 - s) % Z
            comm_ref[s] = comm_ref[s] + x_ref[
                0, pl.ds(ro, MH), pl.ds(recv_j * NP, NP)
            ]

        swap = pltpu.make_async_remote_copy(
            src_ref=comm_ref.at[Z - 2],
            dst_ref=out_ref.at[pl.ds(ro, MH), :],
            send_sem=x_sem_s,
            recv_sem=x_sem_r,
            device_id=(1 - mx, my, mz),
            device_id_type=pl.DeviceIdType.MESH,
        )
        swap.start()
        out_ref[pl.ds(ro, MH), :] = comm_ref[Z - 2]
        swap.wait()

    return pl.pallas_call(
        body,
        out_shape=jax.ShapeDtypeStruct((M, NP), jnp.float32),
        in_specs=[pl.BlockSpec(memory_space=pltpu.VMEM)],
        out_specs=pl.BlockSpec(memory_space=pltpu.VMEM),
        scratch_shapes=[
            pltpu.VMEM((Z - 1, MH, NP), jnp.float32),
            pltpu.SemaphoreType.DMA((Z - 1,)),
            pltpu.SemaphoreType.DMA((Z - 1,)),
            pltpu.SemaphoreType.DMA,
            pltpu.SemaphoreType.DMA,
        ],
        compiler_params=pltpu.CompilerParams(collective_id=0),
    )(x)


# device time: 52180 ns/iter; 1.5663x vs baseline; 1.1747x over previous
import jax
import jax.numpy as jnp
from jax import lax
from jax.experimental import pallas as pl
from jax.experimental.pallas import tpu as pltpu

Z = 4
M = 1024
MH = M // 2
C = 2
SH = MH // C
N = 2048
NP = N // Z


def kernel(x):
    def body(x_ref, out_ref, comm_ref, zsend_sems, zrecv_sems, xsend_sems,
             xrecv_sems):
        mx = lax.axis_index("x")
        my = lax.axis_index("y")
        mz = lax.axis_index("z")
        left = (mz + Z - 1) % Z
        right = (mz + 1) % Z
        ro = mx * MH

        barrier_sem = pltpu.get_barrier_semaphore()
        pl.semaphore_signal(
            barrier_sem, inc=1,
            device_id=(mx, my, left), device_id_type=pl.DeviceIdType.MESH,
        )
        pl.semaphore_signal(
            barrier_sem, inc=1,
            device_id=(mx, my, right), device_id_type=pl.DeviceIdType.MESH,
        )
        pl.semaphore_signal(
            barrier_sem, inc=1,
            device_id=(1 - mx, my, mz), device_id_type=pl.DeviceIdType.MESH,
        )
        pl.semaphore_wait(barrier_sem, 3)

        def z_send(s, c):
            send_j = (mz + Z - 1 - s) % Z
            if s == 0:
                src = x_ref.at[
                    0, pl.ds(ro + c * SH, SH), pl.ds(send_j * NP, NP)
                ]
            else:
                src = comm_ref.at[s - 1, pl.ds(c * SH, SH)]
            rdma = pltpu.make_async_remote_copy(
                src_ref=src,
                dst_ref=comm_ref.at[s, pl.ds(c * SH, SH)],
                send_sem=zsend_sems.at[s, c],
                recv_sem=zrecv_sems.at[s, c],
                device_id=(mx, my, right),
                device_id_type=pl.DeviceIdType.MESH,
            )
            rdma.start()
            return rdma

        def x_swap(c):
            rdma = pltpu.make_async_remote_copy(
                src_ref=comm_ref.at[Z - 2, pl.ds(c * SH, SH)],
                dst_ref=out_ref.at[pl.ds(ro + c * SH, SH), :],
                send_sem=xsend_sems.at[c],
                recv_sem=xrecv_sems.at[c],
                device_id=(1 - mx, my, mz),
                device_id_type=pl.DeviceIdType.MESH,
            )
            rdma.start()
            return rdma

        pending = []
        swaps = []
        for c in range(C):
            pending.append(z_send(0, c))

        for s in range(Z - 1):
            recv_j = (mz + Z - 2 - s) % Z
            for c in range(C):
                pending[s * C + c].wait_recv()
                comm_ref[s, pl.ds(c * SH, SH)] = comm_ref[
                    s, pl.ds(c * SH, SH)
                ] + x_ref[0, pl.ds(ro + c * SH, SH), pl.ds(recv_j * NP, NP)]
                if s < Z - 2:
                    pending.append(z_send(s + 1, c))
                else:
                    swaps.append(x_swap(c))
                    out_ref[pl.ds(ro + c * SH, SH), :] = comm_ref[
                        s, pl.ds(c * SH, SH)
                    ]

        for rdma in pending:
            rdma.wait_send()
        for rdma in swaps:
            rdma.wait_send()
            rdma.wait_recv()

    return pl.pallas_call(
        body,
        out_shape=jax.ShapeDtypeStruct((M, NP), jnp.float32),
        in_specs=[pl.BlockSpec(memory_space=pltpu.VMEM)],
        out_specs=pl.BlockSpec(memory_space=pltpu.VMEM),
        scratch_shapes=[
            pltpu.VMEM((Z - 1, MH, NP), jnp.float32),
            pltpu.SemaphoreType.DMA((Z - 1, C)),
            pltpu.SemaphoreType.DMA((Z - 1, C)),
            pltpu.SemaphoreType.DMA((C,)),
            pltpu.SemaphoreType.DMA((C,)),
        ],
        compiler_params=pltpu.CompilerParams(collective_id=0),
    )(x)


# device time: 39191 ns/iter; 2.0854x vs baseline; 1.3314x over previous
import jax
import jax.numpy as jnp
from jax import lax
from jax.experimental import pallas as pl
from jax.experimental.pallas import tpu as pltpu

Z = 4
M = 1024
MQ = M // 4
C = 2
SH = MQ // C
N = 2048
NP = N // Z


def kernel(x):
    def body(x_ref, out_ref, comm_ref, zsend, zrecv, xsend, xrecv, ysend,
             yrecv):
        mx = lax.axis_index("x")
        my = lax.axis_index("y")
        mz = lax.axis_index("z")
        left = (mz + Z - 1) % Z
        right = (mz + 1) % Z
        q = my % 2
        ypart = my - q + (1 - q)
        ro = q * 512 + mx * MQ
        rx = q * 512 + (1 - mx) * MQ

        barrier_sem = pltpu.get_barrier_semaphore()
        for did in (
            (mx, my, left),
            (mx, my, right),
            (1 - mx, my, mz),
            (mx, ypart, mz),
        ):
            pl.semaphore_signal(
                barrier_sem, inc=1,
                device_id=did, device_id_type=pl.DeviceIdType.MESH,
            )
        pl.semaphore_wait(barrier_sem, 4)

        def z_send(s, c):
            send_j = (mz + Z - 1 - s) % Z
            if s == 0:
                src = x_ref.at[
                    0, pl.ds(ro + c * SH, SH), pl.ds(send_j * NP, NP)
                ]
            else:
                src = comm_ref.at[s - 1, pl.ds(c * SH, SH)]
            rdma = pltpu.make_async_remote_copy(
                src_ref=src,
                dst_ref=comm_ref.at[s, pl.ds(c * SH, SH)],
                send_sem=zsend.at[s, c],
                recv_sem=zrecv.at[s, c],
                device_id=(mx, my, right),
                device_id_type=pl.DeviceIdType.MESH,
            )
            rdma.start()
            return rdma

        def swap_out(src_ref, row0, send_sem, recv_sem, target):
            rdma = pltpu.make_async_remote_copy(
                src_ref=src_ref,
                dst_ref=out_ref.at[pl.ds(row0, SH), :],
                send_sem=send_sem,
                recv_sem=recv_sem,
                device_id=target,
                device_id_type=pl.DeviceIdType.MESH,
            )
            rdma.start()
            return rdma

        pending = []
        drains = []
        xswaps = []
        for c in range(C):
            pending.append(z_send(0, c))

        for s in range(Z - 1):
            recv_j = (mz + Z - 2 - s) % Z
            for c in range(C):
                pending[s * C + c].wait_recv()
                comm_ref[s, pl.ds(c * SH, SH)] = comm_ref[
                    s, pl.ds(c * SH, SH)
                ] + x_ref[0, pl.ds(ro + c * SH, SH), pl.ds(recv_j * NP, NP)]
                if s < Z - 2:
                    pending.append(z_send(s + 1, c))
                else:
                    final = comm_ref.at[s, pl.ds(c * SH, SH)]
                    out_ref[pl.ds(ro + c * SH, SH), :] = comm_ref[
                        s, pl.ds(c * SH, SH)
                    ]
                    xswaps.append(
                        swap_out(final, ro + c * SH, xsend.at[c],
                                 xrecv.at[c], (1 - mx, my, mz))
                    )
                    drains.append(
                        swap_out(final, ro + c * SH, ysend.at[c],
                                 yrecv.at[c], (mx, ypart, mz))
                    )

        for c in range(C):
            xswaps[c].wait_recv()
            drains.append(
                swap_out(
                    out_ref.at[pl.ds(rx + c * SH, SH), :],
                    rx + c * SH,
                    ysend.at[C + c],
                    yrecv.at[C + c],
                    (mx, ypart, mz),
                )
            )

        for rdma in pending + xswaps + drains:
            rdma.wait_send()
        for rdma in drains:
            rdma.wait_recv()

    return pl.pallas_call(
        body,
        out_shape=jax.ShapeDtypeStruct((M, NP), jnp.float32),
        in_specs=[pl.BlockSpec(memory_space=pltpu.VMEM)],
        out_specs=pl.BlockSpec(memory_space=pltpu.VMEM),
        scratch_shapes=[
            pltpu.VMEM((Z - 1, MQ, NP), jnp.float32),
            pltpu.SemaphoreType.DMA((Z - 1, C)),
            pltpu.SemaphoreType.DMA((Z - 1, C)),
            pltpu.SemaphoreType.DMA((C,)),
            pltpu.SemaphoreType.DMA((C,)),
            pltpu.SemaphoreType.DMA((2 * C,)),
            pltpu.SemaphoreType.DMA((2 * C,)),
        ],
        compiler_params=pltpu.CompilerParams(collective_id=0),
    )(x)


# device time: 37999 ns/iter; 2.1508x vs baseline; 1.0314x over previous
import jax
import jax.numpy as jnp
from jax import lax
from jax.experimental import pallas as pl
from jax.experimental.pallas import tpu as pltpu

Z = 4
M = 1024
MQ = M // 4
C = 4
SH = MQ // C
N = 2048
NP = N // Z


def kernel(x):
    def body(x_ref, out_ref, comm_ref, zsend, zrecv, xsend, xrecv, ysend,
             yrecv):
        mx = lax.axis_index("x")
        my = lax.axis_index("y")
        mz = lax.axis_index("z")
        left = (mz + Z - 1) % Z
        right = (mz + 1) % Z
        q = my % 2
        ypart = my - q + (1 - q)
        ro = q * 512 + mx * MQ
        rx = q * 512 + (1 - mx) * MQ

        barrier_sem = pltpu.get_barrier_semaphore()
        for did in (
            (mx, my, left),
            (mx, my, right),
            (1 - mx, my, mz),
            (mx, ypart, mz),
        ):
            pl.semaphore_signal(
                barrier_sem, inc=1,
                device_id=did, device_id_type=pl.DeviceIdType.MESH,
            )
        pl.semaphore_wait(barrier_sem, 4)

        def z_send(s, c):
            send_j = (mz + Z - 1 - s) % Z
            if s == 0:
                src = x_ref.at[
                    0, pl.ds(ro + c * SH, SH), pl.ds(send_j * NP, NP)
                ]
            else:
                src = comm_ref.at[s - 1, pl.ds(c * SH, SH)]
            rdma = pltpu.make_async_remote_copy(
                src_ref=src,
                dst_ref=comm_ref.at[s, pl.ds(c * SH, SH)],
                send_sem=zsend.at[s, c],
                recv_sem=zrecv.at[s, c],
                device_id=(mx, my, right),
                device_id_type=pl.DeviceIdType.MESH,
            )
            rdma.start()
            return rdma

        def swap_out(src_ref, row0, send_sem, recv_sem, target):
            rdma = pltpu.make_async_remote_copy(
                src_ref=src_ref,
                dst_ref=out_ref.at[pl.ds(row0, SH), :],
                send_sem=send_sem,
                recv_sem=recv_sem,
                device_id=target,
                device_id_type=pl.DeviceIdType.MESH,
            )
            rdma.start()
            return rdma

        pending = []
        drains = []
        xswaps = []
        for c in range(C):
            pending.append(z_send(0, c))

        for s in range(Z - 1):
            recv_j = (mz + Z - 2 - s) % Z
            for c in range(C):
                pending[s * C + c].wait_recv()
                comm_ref[s, pl.ds(c * SH, SH)] = comm_ref[
                    s, pl.ds(c * SH, SH)
                ] + x_ref[0, pl.ds(ro + c * SH, SH), pl.ds(recv_j * NP, NP)]
                if s < Z - 2:
                    pending.append(z_send(s + 1, c))
                else:
                    final = comm_ref.at[s, pl.ds(c * SH, SH)]
                    out_ref[pl.ds(ro + c * SH, SH), :] = comm_ref[
                        s, pl.ds(c * SH, SH)
                    ]
                    xswaps.append(
                        swap_out(final, ro + c * SH, xsend.at[c],
                                 xrecv.at[c], (1 - mx, my, mz))
                    )
                    drains.append(
                        swap_out(final, ro + c * SH, ysend.at[c],
                                 yrecv.at[c], (mx, ypart, mz))
                    )

        for c in range(C):
            xswaps[c].wait_recv()
            drains.append(
                swap_out(
                    out_ref.at[pl.ds(rx + c * SH, SH), :],
                    rx + c * SH,
                    ysend.at[C + c],
                    yrecv.at[C + c],
                    (mx, ypart, mz),
                )
            )

        for rdma in pending + xswaps + drains:
            rdma.wait_send()
        for rdma in drains:
            rdma.wait_recv()

    return pl.pallas_call(
        body,
        out_shape=jax.ShapeDtypeStruct((M, NP), jnp.float32),
        in_specs=[pl.BlockSpec(memory_space=pltpu.VMEM)],
        out_specs=pl.BlockSpec(memory_space=pltpu.VMEM),
        scratch_shapes=[
            pltpu.VMEM((Z - 1, MQ, NP), jnp.float32),
            pltpu.SemaphoreType.DMA((Z - 1, C)),
            pltpu.SemaphoreType.DMA((Z - 1, C)),
            pltpu.SemaphoreType.DMA((C,)),
            pltpu.SemaphoreType.DMA((C,)),
            pltpu.SemaphoreType.DMA((2 * C,)),
            pltpu.SemaphoreType.DMA((2 * C,)),
        ],
        compiler_params=pltpu.CompilerParams(collective_id=0),
    )(x)


# device time: 37735 ns/iter; 2.1659x vs baseline; 1.0070x over previous
import jax
import jax.numpy as jnp
from jax import lax
from jax.experimental import pallas as pl
from jax.experimental.pallas import tpu as pltpu

Z = 4
M = 1024
MQ = M // 4
C = 4
SH = MQ // C
N = 2048
NP = N // Z


def kernel(x):
    def body(x_ref, out_ref, comm_ref, zsend, zrecv, fsend, frecv):
        mx = lax.axis_index("x")
        my = lax.axis_index("y")
        mz = lax.axis_index("z")
        left = (mz + Z - 1) % Z
        right = (mz + 1) % Z
        q = my % 2
        ypart = my - q + (1 - q)
        ro = q * 512 + mx * MQ
        fan = (
            (1 - mx, my, mz),
            (mx, ypart, mz),
            (1 - mx, ypart, mz),
        )

        barrier_sem = pltpu.get_barrier_semaphore()
        for did in ((mx, my, left), (mx, my, right)) + fan:
            pl.semaphore_signal(
                barrier_sem, inc=1,
                device_id=did, device_id_type=pl.DeviceIdType.MESH,
            )
        pl.semaphore_wait(barrier_sem, 5)

        def z_send(s, c):
            send_j = (mz + Z - 1 - s) % Z
            if s == 0:
                src = x_ref.at[
                    0, pl.ds(ro + c * SH, SH), pl.ds(send_j * NP, NP)
                ]
            else:
                src = comm_ref.at[s - 1, pl.ds(c * SH, SH)]
            rdma = pltpu.make_async_remote_copy(
                src_ref=src,
                dst_ref=comm_ref.at[s, pl.ds(c * SH, SH)],
                send_sem=zsend.at[s, c],
                recv_sem=zrecv.at[s, c],
                device_id=(mx, my, right),
                device_id_type=pl.DeviceIdType.MESH,
            )
            rdma.start()
            return rdma

        def swap_out(src_ref, row0, send_sem, recv_sem, target):
            rdma = pltpu.make_async_remote_copy(
                src_ref=src_ref,
                dst_ref=out_ref.at[pl.ds(row0, SH), :],
                send_sem=send_sem,
                recv_sem=recv_sem,
                device_id=target,
                device_id_type=pl.DeviceIdType.MESH,
            )
            rdma.start()
            return rdma

        pending = []
        drains = []
        for c in range(C):
            pending.append(z_send(0, c))

        for s in range(Z - 1):
            recv_j = (mz + Z - 2 - s) % Z
            for c in range(C):
                pending[s * C + c].wait_recv()
                comm_ref[s, pl.ds(c * SH, SH)] = comm_ref[
                    s, pl.ds(c * SH, SH)
                ] + x_ref[0, pl.ds(ro + c * SH, SH), pl.ds(recv_j * NP, NP)]
                if s < Z - 2:
                    pending.append(z_send(s + 1, c))
                else:
                    final = comm_ref.at[s, pl.ds(c * SH, SH)]
                    out_ref[pl.ds(ro + c * SH, SH), :] = comm_ref[
                        s, pl.ds(c * SH, SH)
                    ]
                    for k, target in enumerate(fan):
                        drains.append(
                            swap_out(final, ro + c * SH, fsend.at[k, c],
                                     frecv.at[k, c], target)
                        )

        for rdma in pending + drains:
            rdma.wait_send()
        for rdma in drains:
            rdma.wait_recv()

    return pl.pallas_call(
        body,
        out_shape=jax.ShapeDtypeStruct((M, NP), jnp.float32),
        in_specs=[pl.BlockSpec(memory_space=pltpu.VMEM)],
        out_specs=pl.BlockSpec(memory_space=pltpu.VMEM),
        scratch_shapes=[
            pltpu.VMEM((Z - 1, MQ, NP), jnp.float32),
            pltpu.SemaphoreType.DMA((Z - 1, C)),
            pltpu.SemaphoreType.DMA((Z - 1, C)),
            pltpu.SemaphoreType.DMA((3, C)),
            pltpu.SemaphoreType.DMA((3, C)),
        ],
        compiler_params=pltpu.CompilerParams(collective_id=0),
    )(x)
